# Initial kernel scaffold; baseline (speedup 1.0000x reference)
import jax
import jax.numpy as jnp
from jax.experimental import pallas as pl


def kernel(inputs, onehot_table):
    raise NotImplementedError("write your pallas kernel here")


# speedup vs baseline: 3.5462x; 3.5462x over previous
import functools

import jax
import jax.numpy as jnp
from jax import lax
from jax.experimental import pallas as pl
from jax.experimental.pallas import tpu as pltpu
from jax.experimental.pallas import tpu_sc as plsc

NC = 2
NS = 16
L = 16
NW = NC * NS

B = 4096 * 50
D = 128
NB = B // NW
C = 400
NCHUNK = NB // C
CV = C // L


def _onehot_body(idx_hbm, zeros_hbm, out_hbm, idx_v, buf0, buf1, sem0, sem1):
    wid = lax.axis_index("s") * NC + lax.axis_index("c")
    base = wid * NB

    pltpu.sync_copy(idx_hbm.at[pl.ds(base, NB)], idx_v)
    pltpu.sync_copy(zeros_hbm, buf0)
    pltpu.sync_copy(zeros_hbm, buf1)

    iota = lax.iota(jnp.int32, L)
    ones = jnp.ones((L,), jnp.float32)
    zeros = jnp.zeros((L,), jnp.float32)
    bufs = (buf0, buf1)
    sems = (sem0, sem1)

    def scatter(buf, chunk, vals):
        off = chunk * C

        def body(j, carry):
            cols = idx_v[pl.ds(off + j * L, L)]
            flat = (j * L + iota) * D + cols
            plsc.store_scatter(buf, [flat], vals)
            return carry

        lax.fori_loop(0, CV, body, 0)

    handles = [None, None]
    for k in range(NCHUNK):
        b = k % 2
        if handles[b] is not None:
            handles[b].wait()
            scatter(bufs[b], k - 2, zeros)
        scatter(bufs[b], k, ones)
        handles[b] = pltpu.async_copy(
            bufs[b], out_hbm.at[pl.ds((base + k * C) * D, C * D)], sems[b]
        )
    handles[0].wait()
    handles[1].wait()


_onehot_call = pl.kernel(
    _onehot_body,
    out_type=jax.ShapeDtypeStruct((B * D,), jnp.float32),
    mesh=plsc.VectorSubcoreMesh(
        core_axis_name="c", subcore_axis_name="s", num_cores=NC, num_subcores=NS
    ),
    scratch_types=[
        pltpu.VMEM((NB,), jnp.int32),
        pltpu.VMEM((C * D,), jnp.float32),
        pltpu.VMEM((C * D,), jnp.float32),
        pltpu.SemaphoreType.DMA,
        pltpu.SemaphoreType.DMA,
    ],
    compiler_params=pltpu.CompilerParams(needs_layout_passes=False),
)


@jax.jit
def kernel(inputs, onehot_table):
    del onehot_table
    idx = inputs.reshape(-1).astype(jnp.int32)
    zeros = jnp.zeros((C * D,), jnp.float32)
    out = _onehot_call(idx, zeros)
    return out.reshape(inputs.shape[0], inputs.shape[1], D)


# speedup vs baseline: 6.8047x; 1.9189x over previous
import functools

import jax
import jax.numpy as jnp
from jax import lax
from jax.experimental import pallas as pl
from jax.experimental.pallas import tpu as pltpu
from jax.experimental.pallas import tpu_sc as plsc

NC = 2
NS = 16
L = 16
NW = NC * NS

N0 = 4096
N1 = 50
D = 128
B = N0 * N1
NB = B // NW
S = 8
C = S * N1
NCHUNK = NB // C
CV = C // L
N1P = 56


def _onehot_body(idx_hbm, zeros_hbm, out_hbm, idx_v, buf0, buf1, sem0, sem1):
    wid = lax.axis_index("s") * NC + lax.axis_index("c")
    base = wid * NB

    pltpu.sync_copy(idx_hbm.at[pl.ds(base, NB)], idx_v)
    pltpu.sync_copy(zeros_hbm, buf0)
    pltpu.sync_copy(zeros_hbm, buf1)

    iota = lax.iota(jnp.int32, L)
    ones = jnp.ones((L,), jnp.float32)
    zeros = jnp.zeros((L,), jnp.float32)
    bufs = (buf0, buf1)
    sems = (sem0, sem1)

    def scatter(buf, chunk, vals):
        off = chunk * C

        def body(j, carry):
            r = j * L + iota
            cols = idx_v[pl.ds(off + j * L, L)]
            plsc.store_scatter(buf, [r // N1, r % N1, cols], vals)
            return carry

        lax.fori_loop(0, CV, body, 0)

    handles = [None, None]
    for k in range(NCHUNK):
        b = k % 2
        if handles[b] is not None:
            handles[b].wait()
            scatter(bufs[b], k - 2, zeros)
        scatter(bufs[b], k, ones)
        i0 = wid * (NCHUNK * S) + k * S
        handles[b] = pltpu.async_copy(
            bufs[b].at[:, pl.ds(0, N1), :], out_hbm.at[pl.ds(i0, S)], sems[b]
        )
    handles[0].wait()
    handles[1].wait()


_onehot_call = pl.kernel(
    _onehot_body,
    out_type=jax.ShapeDtypeStruct((N0, N1, D), jnp.float32),
    mesh=plsc.VectorSubcoreMesh(
        core_axis_name="c", subcore_axis_name="s", num_cores=NC, num_subcores=NS
    ),
    scratch_types=[
        pltpu.VMEM((NB,), jnp.int32),
        pltpu.VMEM((S, N1P, D), jnp.float32),
        pltpu.VMEM((S, N1P, D), jnp.float32),
        pltpu.SemaphoreType.DMA,
        pltpu.SemaphoreType.DMA,
    ],
    compiler_params=pltpu.CompilerParams(needs_layout_passes=False),
)


@jax.jit
def kernel(inputs, onehot_table):
    del onehot_table
    idx = inputs.reshape(-1).astype(jnp.int32)
    zeros = jnp.zeros((S, N1P, D), jnp.float32)
    return _onehot_call(idx, zeros)
